# baseline (device time: 15562 ns/iter reference)
import jax
import jax.numpy as jnp
from jax import lax
from jax.experimental import pallas as pl
from jax.experimental.pallas import tpu as pltpu

T = 512
D = 1024
V_LOCAL = 8192
N_DEV = 16
N_REP = 8
V_SUB = V_LOCAL // N_REP


def _coords(idx):
    return idx // 8, (idx // 4) % 2, idx % 4


def kernel(x, W, labels):
    def body(x_hbm, w_hbm, labels_ref, out_ref,
             xv_ref, wv_ref, stats_ref, gather_ref, copy_sem, send_sems,
             recv_sems):
        my_x = lax.axis_index("x")
        my_y = lax.axis_index("y")
        my_z = lax.axis_index("z")
        me = my_x * 8 + my_y * 4 + my_z
        r = my_x * 4 + my_z

        H = V_SUB // 2

        def half_copy(h):
            return pltpu.make_async_copy(
                w_hbm.at[:, pl.ds(r * V_SUB + h * H, H)],
                wv_ref.at[:, pl.ds(h * H, H)],
                copy_sem.at[h],
            )

        x_cp = pltpu.make_async_copy(x_hbm, xv_ref, copy_sem.at[2])
        x_cp.start()
        half_copy(0).start()
        half_copy(1).start()

        barrier_sem = pltpu.get_barrier_semaphore()
        for idx in range(N_DEV):
            @pl.when(me != idx)
            def _():
                pl.semaphore_signal(
                    barrier_sem, inc=1,
                    device_id=_coords(idx),
                    device_id_type=pl.DeviceIdType.MESH,
                )

        ids = lax.broadcasted_iota(jnp.int32, (T, V_SUB // 2), 1)
        local_idx = labels_ref[...] - my_y * V_LOCAL - r * V_SUB
        li2d = local_idx[:, None]
        s = jnp.zeros((T,), jnp.float32)
        p = jnp.zeros((T,), jnp.float32)
        x_cp.wait()
        for h in range(2):
            half_copy(h).wait()
            logits = jnp.dot(xv_ref[...], wv_ref[:, h * H:(h + 1) * H],
                             preferred_element_type=jnp.float32)
            s = s + jnp.sum(jnp.exp(logits), axis=1)
            p = p + jnp.sum(
                jnp.where(ids == li2d - h * H, logits, 0.0),
                axis=1)

        stats_ref[0, :] = s
        stats_ref[1, :] = p
        for idx in range(N_DEV):
            @pl.when(me == idx)
            def _():
                gather_ref[idx, 0, :] = s
                gather_ref[idx, 1, :] = p

        pl.semaphore_wait(barrier_sem, N_DEV - 1)

        def rdma_to(idx):
            return pltpu.make_async_remote_copy(
                src_ref=stats_ref,
                dst_ref=gather_ref.at[me],
                send_sem=send_sems.at[idx],
                recv_sem=recv_sems.at[me],
                device_id=_coords(idx),
                device_id_type=pl.DeviceIdType.MESH,
            )

        for idx in range(N_DEV):
            @pl.when(me != idx)
            def _():
                rdma_to(idx).start()

        for idx in range(N_DEV):
            @pl.when(me != idx)
            def _():
                pltpu.make_async_remote_copy(
                    src_ref=stats_ref,
                    dst_ref=gather_ref.at[idx],
                    send_sem=send_sems.at[idx],
                    recv_sem=recv_sems.at[idx],
                    device_id=_coords(idx),
                    device_id_type=pl.DeviceIdType.MESH,
                ).wait_recv()

        total = jnp.sum(gather_ref[...], axis=0)
        out_ref[...] = jnp.log(total[0, :]) - total[1, :]

        for idx in range(N_DEV):
            @pl.when(me != idx)
            def _():
                rdma_to(idx).wait_send()

    return pl.pallas_call(
        body,
        out_shape=jax.ShapeDtypeStruct((T,), jnp.float32),
        in_specs=[
            pl.BlockSpec(memory_space=pl.ANY),
            pl.BlockSpec(memory_space=pl.ANY),
            pl.BlockSpec(memory_space=pltpu.MemorySpace.VMEM),
        ],
        out_specs=pl.BlockSpec(memory_space=pltpu.MemorySpace.VMEM),
        scratch_shapes=[
            pltpu.VMEM((T, D), jnp.float32),
            pltpu.VMEM((D, V_SUB), jnp.float32),
            pltpu.VMEM((2, T), jnp.float32),
            pltpu.VMEM((N_DEV, 2, T), jnp.float32),
            pltpu.SemaphoreType.DMA((3,)),
            pltpu.SemaphoreType.DMA((N_DEV,)),
            pltpu.SemaphoreType.DMA((N_DEV,)),
        ],
        compiler_params=pltpu.CompilerParams(
            collective_id=0,
            vmem_limit_bytes=100 * 1024 * 1024,
        ),
    )(x, W, labels)
